# Initial kernel scaffold; baseline (speedup 1.0000x reference)
import jax
import jax.numpy as jnp
from jax.experimental import pallas as pl


def kernel(h_c, h_p, edge_index, Wq_c, Wq_p, Wk, Wv):
    raise NotImplementedError("write your pallas kernel here")


# speedup vs baseline: 65.0758x; 65.0758x over previous
import functools

import jax
import jax.numpy as jnp
from jax import lax
from jax.experimental import pallas as pl
from jax.experimental.pallas import tpu as pltpu
from jax.experimental.pallas import tpu_sc as plsc

N = 10000
E = 320000
IN_DIM = 128
DIM = 128
NUM_HEAD = 8
HEAD_DIM = DIM // NUM_HEAD
SCALE = 1.0 / DIM ** 0.5

NC = 2
NS = 16
HGRP = 4
HCOLS = HGRP * HEAD_DIM

EPS = E // NS
C = 80
NCHUNK = EPS // C
RPT = 624
ZROWS = 48
TAIL = N - NS * RPT


def _proj_body(hc_ref, hp_ref, wqc_ref, wqp_ref, wk_ref, wv_ref,
               qt_ref, kvt_ref):
    hc = hc_ref[...]
    hp = hp_ref[...]
    qc = jnp.dot(hc, wqc_ref[...], preferred_element_type=jnp.float32) * SCALE
    qp = jnp.dot(hp, wqp_ref[...], preferred_element_type=jnp.float32) * SCALE
    k = jnp.dot(hc, wk_ref[...], preferred_element_type=jnp.float32)
    v = jnp.dot(hc, wv_ref[...], preferred_element_type=jnp.float32)
    qt_ref[0] = jnp.concatenate([qc[:, :HCOLS], qp[:, :HCOLS]], axis=1)
    qt_ref[1] = jnp.concatenate([qc[:, HCOLS:], qp[:, HCOLS:]], axis=1)
    kvt_ref[0] = jnp.concatenate([k[:, :HCOLS], v[:, :HCOLS]], axis=1)
    kvt_ref[1] = jnp.concatenate([k[:, HCOLS:], v[:, HCOLS:]], axis=1)


def _project(h_c, h_p, Wq_c, Wq_p, Wk, Wv):
    R = 1000
    grid = (N // R,)
    in_specs = [
        pl.BlockSpec((R, IN_DIM), lambda i: (i, 0)),
        pl.BlockSpec((R, IN_DIM), lambda i: (i, 0)),
    ] + [pl.BlockSpec((IN_DIM, DIM), lambda i: (0, 0))] * 4
    out_specs = [pl.BlockSpec((NC, R, DIM), lambda i: (0, i, 0))] * 2
    out_shape = [jax.ShapeDtypeStruct((NC, N, DIM), jnp.float32)] * 2
    return pl.pallas_call(
        _proj_body,
        grid=grid,
        in_specs=in_specs,
        out_specs=out_specs,
        out_shape=out_shape,
    )(h_c, h_p, Wq_c, Wq_p, Wk, Wv)


def _edge_body(src_ref, dst_ref, qt_ref, kvt_ref, out_ref,
               src_v, dst_v, qv, kvv, ov, zb, acc, sem):
    cid = lax.axis_index("c")
    sid = lax.axis_index("s")

    zeros16 = jnp.zeros((16,), jnp.float32)

    def zrow(r, carry):
        for j in range(DIM // 16):
            zb[r, pl.ds(j * 16, 16)] = zeros16
        return carry

    lax.fori_loop(0, ZROWS, zrow, 0)
    row0 = sid * RPT
    for j in range(RPT // ZROWS):
        pltpu.sync_copy(zb, acc.at[pl.ds(row0 + j * ZROWS, ZROWS)])

    @pl.when(sid == NS - 1)
    def _zero_tail():
        pltpu.sync_copy(zb.at[pl.ds(0, TAIL)], acc.at[pl.ds(NS * RPT, TAIL)])

    plsc.subcore_barrier()

    qt = qt_ref.at[cid]
    kvt = kvt_ref.at[cid]

    lanes = lax.iota(jnp.int32, 16)
    perms = [lanes ^ st for st in (1, 2, 4, 8)]
    dnums = lax.GatherDimensionNumbers(
        offset_dims=(), collapsed_slice_dims=(0,), start_index_map=(0,))

    def lane_sum(x):
        for p in perms:
            x = x + lax.gather(
                x, p[:, None], dnums, (1,),
                mode=lax.GatherScatterMode.PROMISE_IN_BOUNDS)
        return x

    def chunk_body(ci, carry):
        base = sid * EPS + ci * C
        pltpu.sync_copy(src_ref.at[pl.ds(base, C)], src_v)
        pltpu.sync_copy(dst_ref.at[pl.ds(base, C)], dst_v)
        cp1 = pltpu.async_copy(kvt.at[src_v], kvv, sem)
        cp2 = pltpu.async_copy(qt.at[dst_v], qv, sem)
        cp1.wait()
        cp2.wait()

        def edge_one(e, inner):
            for h in range(HGRP):
                sl = pl.ds(h * HEAD_DIM, HEAD_DIM)
                slv = pl.ds(HCOLS + h * HEAD_DIM, HEAD_DIM)
                q_c = qv[e, sl]
                q_p = qv[e, slv]
                kk = kvv[e, sl]
                vvh = kvv[e, slv]
                w_c = jnp.exp(jnp.clip(lane_sum(q_c * kk), -5.0, 5.0))
                w_p = jnp.exp(jnp.clip(lane_sum(q_p * kk), -5.0, 5.0))
                ov[e, sl] = vvh * w_c
                ov[e, slv] = vvh * w_p
            return inner

        lax.fori_loop(0, C, edge_one, 0)
        pltpu.sync_copy(ov, acc.at[dst_v], add=True)
        return carry

    lax.fori_loop(0, NCHUNK, chunk_body, 0)
    plsc.subcore_barrier()

    pltpu.sync_copy(acc.at[pl.ds(row0, RPT)], out_ref.at[cid, pl.ds(row0, RPT)])

    @pl.when(sid == NS - 1)
    def _flush_tail():
        pltpu.sync_copy(acc.at[pl.ds(NS * RPT, TAIL)],
                        out_ref.at[cid, pl.ds(NS * RPT, TAIL)])


def _edge_attn(edge_index, qt, kvt):
    mesh = plsc.VectorSubcoreMesh(core_axis_name="c", subcore_axis_name="s")
    f = functools.partial(
        pl.kernel,
        out_type=jax.ShapeDtypeStruct((NC, N, DIM), jnp.float32),
        mesh=mesh,
        scratch_types=[
            pltpu.VMEM((C,), jnp.int32),
            pltpu.VMEM((C,), jnp.int32),
            pltpu.VMEM((C, DIM), jnp.float32),
            pltpu.VMEM((C, DIM), jnp.float32),
            pltpu.VMEM((C, DIM), jnp.float32),
            pltpu.VMEM((ZROWS, DIM), jnp.float32),
            pltpu.VMEM_SHARED((N, DIM), jnp.float32),
            pltpu.SemaphoreType.DMA,
        ],
    )(_edge_body)
    return f(edge_index[0], edge_index[1], qt, kvt)


def kernel(h_c, h_p, edge_index, Wq_c, Wq_p, Wk, Wv):
    qt, kvt = _project(h_c, h_p, Wq_c, Wq_p, Wk, Wv)
    o2 = _edge_attn(edge_index, qt, kvt)
    out_c = jnp.concatenate([o2[0, :, :HCOLS], o2[1, :, :HCOLS]], axis=1)
    out_p = jnp.concatenate([o2[0, :, HCOLS:], o2[1, :, HCOLS:]], axis=1)
    return out_c, out_p


# speedup vs baseline: 73.4880x; 1.1293x over previous
import functools

import jax
import jax.numpy as jnp
from jax import lax
from jax.experimental import pallas as pl
from jax.experimental.pallas import tpu as pltpu
from jax.experimental.pallas import tpu_sc as plsc

N = 10000
E = 320000
IN_DIM = 128
DIM = 128
NUM_HEAD = 8
HEAD_DIM = DIM // NUM_HEAD
SCALE = 1.0 / DIM ** 0.5

NC = 2
NS = 16
HGRP = 4
HCOLS = HGRP * HEAD_DIM

EPS = E // NS
C = 40
NCHUNK = EPS // C
RPT = 624
ZROWS = 16
TAIL = N - NS * RPT


def _proj_body(hc_ref, hp_ref, wqc_ref, wqp_ref, wk_ref, wv_ref,
               qt_ref, kvt_ref):
    hc = hc_ref[...]
    hp = hp_ref[...]
    qc = jnp.dot(hc, wqc_ref[...], preferred_element_type=jnp.float32) * SCALE
    qp = jnp.dot(hp, wqp_ref[...], preferred_element_type=jnp.float32) * SCALE
    k = jnp.dot(hc, wk_ref[...], preferred_element_type=jnp.float32)
    v = jnp.dot(hc, wv_ref[...], preferred_element_type=jnp.float32)
    qt_ref[0] = jnp.concatenate([qc[:, :HCOLS], qp[:, :HCOLS]], axis=1)
    qt_ref[1] = jnp.concatenate([qc[:, HCOLS:], qp[:, HCOLS:]], axis=1)
    kvt_ref[0] = jnp.concatenate([k[:, :HCOLS], v[:, :HCOLS]], axis=1)
    kvt_ref[1] = jnp.concatenate([k[:, HCOLS:], v[:, HCOLS:]], axis=1)


def _project(h_c, h_p, Wq_c, Wq_p, Wk, Wv):
    R = 1000
    grid = (N // R,)
    in_specs = [
        pl.BlockSpec((R, IN_DIM), lambda i: (i, 0)),
        pl.BlockSpec((R, IN_DIM), lambda i: (i, 0)),
    ] + [pl.BlockSpec((IN_DIM, DIM), lambda i: (0, 0))] * 4
    out_specs = [pl.BlockSpec((NC, R, DIM), lambda i: (0, i, 0))] * 2
    out_shape = [jax.ShapeDtypeStruct((NC, N, DIM), jnp.float32)] * 2
    return pl.pallas_call(
        _proj_body,
        grid=grid,
        in_specs=in_specs,
        out_specs=out_specs,
        out_shape=out_shape,
    )(h_c, h_p, Wq_c, Wq_p, Wk, Wv)


def _edge_body(src_ref, dst_ref, qt_ref, kvt_ref, out_ref,
               src_va, dst_va, src_vb, dst_vb,
               qva, kvva, qvb, kvvb, ov, zb, acc, sema, semb):
    cid = lax.axis_index("c")
    sid = lax.axis_index("s")

    zeros16 = jnp.zeros((16,), jnp.float32)

    def zrow(r, carry):
        for j in range(DIM // 16):
            zb[r, pl.ds(j * 16, 16)] = zeros16
        return carry

    lax.fori_loop(0, ZROWS, zrow, 0)
    row0 = sid * RPT
    for j in range(RPT // ZROWS):
        pltpu.sync_copy(zb, acc.at[pl.ds(row0 + j * ZROWS, ZROWS)])

    @pl.when(sid == NS - 1)
    def _zero_tail():
        pltpu.sync_copy(zb.at[pl.ds(0, TAIL)], acc.at[pl.ds(NS * RPT, TAIL)])

    plsc.subcore_barrier()

    qt = qt_ref.at[cid]
    kvt = kvt_ref.at[cid]

    lanes = lax.iota(jnp.int32, 16)
    perms = [lanes ^ st for st in (1, 2, 4, 8)]
    dnums = lax.GatherDimensionNumbers(
        offset_dims=(), collapsed_slice_dims=(0,), start_index_map=(0,))

    def lane_sum(x):
        for p in perms:
            x = x + lax.gather(
                x, p[:, None], dnums, (1,),
                mode=lax.GatherScatterMode.PROMISE_IN_BOUNDS)
        return x

    def load_idx(ci, sv, dv):
        base = sid * EPS + ci * C
        pltpu.sync_copy(src_ref.at[pl.ds(base, C)], sv)
        pltpu.sync_copy(dst_ref.at[pl.ds(base, C)], dv)

    def start_gathers(sv, dv, kvb, qb, sem):
        pltpu.async_copy(kvt.at[sv], kvb, sem)
        pltpu.async_copy(qt.at[dv], qb, sem)

    def wait_gathers(sv, dv, kvb, qb, sem):
        pltpu.make_async_copy(kvt.at[sv], kvb, sem).wait()
        pltpu.make_async_copy(qt.at[dv], qb, sem).wait()

    def compute_scatter(qb, kvb, dv):
        def edge_one(e, inner):
            for h in range(HGRP):
                sl = pl.ds(h * HEAD_DIM, HEAD_DIM)
                slv = pl.ds(HCOLS + h * HEAD_DIM, HEAD_DIM)
                q_c = qb[e, sl]
                q_p = qb[e, slv]
                kk = kvb[e, sl]
                vvh = kvb[e, slv]
                w_c = jnp.exp(jnp.clip(lane_sum(q_c * kk), -5.0, 5.0))
                w_p = jnp.exp(jnp.clip(lane_sum(q_p * kk), -5.0, 5.0))
                ov[e, sl] = vvh * w_c
                ov[e, slv] = vvh * w_p
            return inner

        lax.fori_loop(0, C, edge_one, 0)
        pltpu.sync_copy(ov, acc.at[dv], add=True)

    load_idx(0, src_va, dst_va)
    start_gathers(src_va, dst_va, kvva, qva, sema)

    def chunk_pair(i2, carry):
        n = 2 * i2
        load_idx(n + 1, src_vb, dst_vb)
        start_gathers(src_vb, dst_vb, kvvb, qvb, semb)
        wait_gathers(src_va, dst_va, kvva, qva, sema)
        compute_scatter(qva, kvva, dst_va)

        @pl.when(i2 < NCHUNK // 2 - 1)
        def _prefetch_a():
            load_idx(n + 2, src_va, dst_va)
            start_gathers(src_va, dst_va, kvva, qva, sema)

        wait_gathers(src_vb, dst_vb, kvvb, qvb, semb)
        compute_scatter(qvb, kvvb, dst_vb)
        return carry

    lax.fori_loop(0, NCHUNK // 2, chunk_pair, 0)
    plsc.subcore_barrier()

    pltpu.sync_copy(acc.at[pl.ds(row0, RPT)], out_ref.at[cid, pl.ds(row0, RPT)])

    @pl.when(sid == NS - 1)
    def _flush_tail():
        pltpu.sync_copy(acc.at[pl.ds(NS * RPT, TAIL)],
                        out_ref.at[cid, pl.ds(NS * RPT, TAIL)])


def _edge_attn(edge_index, qt, kvt):
    mesh = plsc.VectorSubcoreMesh(core_axis_name="c", subcore_axis_name="s")
    f = functools.partial(
        pl.kernel,
        out_type=jax.ShapeDtypeStruct((NC, N, DIM), jnp.float32),
        mesh=mesh,
        scratch_types=[
            pltpu.VMEM((C,), jnp.int32),
            pltpu.VMEM((C,), jnp.int32),
            pltpu.VMEM((C,), jnp.int32),
            pltpu.VMEM((C,), jnp.int32),
            pltpu.VMEM((C, DIM), jnp.float32),
            pltpu.VMEM((C, DIM), jnp.float32),
            pltpu.VMEM((C, DIM), jnp.float32),
            pltpu.VMEM((C, DIM), jnp.float32),
            pltpu.VMEM((C, DIM), jnp.float32),
            pltpu.VMEM((ZROWS, DIM), jnp.float32),
            pltpu.VMEM_SHARED((N, DIM), jnp.float32),
            pltpu.SemaphoreType.DMA,
            pltpu.SemaphoreType.DMA,
        ],
    )(_edge_body)
    return f(edge_index[0], edge_index[1], qt, kvt)


def kernel(h_c, h_p, edge_index, Wq_c, Wq_p, Wk, Wv):
    qt, kvt = _project(h_c, h_p, Wq_c, Wq_p, Wk, Wv)
    o2 = _edge_attn(edge_index, qt, kvt)
    out_c = jnp.concatenate([o2[0, :, :HCOLS], o2[1, :, :HCOLS]], axis=1)
    out_p = jnp.concatenate([o2[0, :, HCOLS:], o2[1, :, HCOLS:]], axis=1)
    return out_c, out_p
